# baseline (speedup 1.0000x reference)
import functools

import jax
import jax.numpy as jnp
from jax import lax
from jax.experimental import pallas as pl
from jax.experimental.pallas import tpu as pltpu
from jax.experimental.pallas import tpu_sc as plsc

N_NODES = 10000
N_EDGES = 320000
D = 128
B_GRAPHS = 64

NC, NS = 2, 16
NW = NC * NS
WIN = 128
NWIN = 80
NSTAGE = 5
WPS = NWIN // NSTAGE
EDGES_PER_TILE = WIN * NWIN
E_PAD = EDGES_PER_TILE * NW
NPAD = 10112
ROWS_PER_SUB = NPAD // NS
R = 1000
GRID = N_NODES // R


def _sc_mesh():
    return plsc.VectorSubcoreMesh(
        core_axis_name="c", subcore_axis_name="s", num_cores=NC, num_subcores=NS
    )


def _sc_agg_body(x_hbm, src_hbm, dst_hbm, z2d_hbm, agg_out,
                 agg_sh, srcv, dstv, buf_a, buf_b,
                 sem_a, sem_b, sem_sa, sem_sb):
    c = lax.axis_index("c")
    s = lax.axis_index("s")
    w = s * NC + c
    row0 = s * ROWS_PER_SUB

    pltpu.sync_copy(z2d_hbm.at[pl.ds(row0, ROWS_PER_SUB)],
                    agg_sh.at[pl.ds(row0, ROWS_PER_SUB)])
    plsc.subcore_barrier()

    for t in range(NSTAGE):
        wrow = w * NWIN + t * WPS
        pltpu.sync_copy(src_hbm.at[pl.ds(wrow, WPS)], srcv)
        pltpu.sync_copy(dst_hbm.at[pl.ds(wrow, WPS)], dstv)

        pltpu.async_copy(x_hbm.at[srcv.at[0]], buf_a, sem_a)
        pltpu.async_copy(x_hbm.at[srcv.at[1]], buf_b, sem_b)

        @pl.loop(0, WPS, step=2)
        def _(j):
            pltpu.make_async_copy(x_hbm.at[srcv.at[j]], buf_a, sem_a).wait()
            pltpu.async_copy(buf_a, agg_sh.at[dstv.at[j]], sem_sa, add=True)

            pltpu.make_async_copy(x_hbm.at[srcv.at[j + 1]], buf_b, sem_b).wait()
            pltpu.async_copy(buf_b, agg_sh.at[dstv.at[j + 1]], sem_sb, add=True)

            pltpu.make_async_copy(buf_a, agg_sh.at[dstv.at[j]], sem_sa).wait()

            @pl.when(j + 2 < WPS)
            def _():
                pltpu.async_copy(x_hbm.at[srcv.at[j + 2]], buf_a, sem_a)

            pltpu.make_async_copy(
                buf_b, agg_sh.at[dstv.at[j + 1]], sem_sb).wait()

            @pl.when(j + 3 < WPS)
            def _():
                pltpu.async_copy(x_hbm.at[srcv.at[j + 3]], buf_b, sem_b)

    plsc.subcore_barrier()

    pltpu.sync_copy(agg_sh.at[pl.ds(row0, ROWS_PER_SUB)],
                    agg_out.at[c, pl.ds(row0, ROWS_PER_SUB)])


def _make_sc_agg():
    return pl.kernel(
        _sc_agg_body,
        out_type=jax.ShapeDtypeStruct((NC, NPAD, D), jnp.float32),
        mesh=_sc_mesh(),
        scratch_types=[
            pltpu.VMEM_SHARED((NPAD, D), jnp.float32),
            pltpu.VMEM((WPS, WIN), jnp.int32),
            pltpu.VMEM((WPS, WIN), jnp.int32),
            pltpu.VMEM((WIN, D), jnp.float32),
            pltpu.VMEM((WIN, D), jnp.float32),
            pltpu.SemaphoreType.DMA,
            pltpu.SemaphoreType.DMA,
            pltpu.SemaphoreType.DMA,
            pltpu.SemaphoreType.DMA,
        ],
    )


def _sc_deg_body(dst_hbm, z2d_hbm, ones_hbm, deg_out,
                 deg_sh, dstv, ones_v, sem):
    c = lax.axis_index("c")
    s = lax.axis_index("s")
    w = s * NC + c
    row0 = s * ROWS_PER_SUB

    pltpu.sync_copy(z2d_hbm.at[pl.ds(row0, ROWS_PER_SUB)],
                    deg_sh.at[pl.ds(row0, ROWS_PER_SUB)])
    pltpu.sync_copy(ones_hbm, ones_v)
    plsc.subcore_barrier()

    for t in range(NSTAGE):
        wrow = w * NWIN + t * WPS
        pltpu.sync_copy(dst_hbm.at[pl.ds(wrow, WPS)], dstv)

        pltpu.async_copy(ones_v, deg_sh.at[dstv.at[0]], sem, add=True)

        @pl.loop(1, WPS)
        def _(j):
            pltpu.async_copy(ones_v, deg_sh.at[dstv.at[j]], sem, add=True)
            pltpu.make_async_copy(ones_v, deg_sh.at[dstv.at[j - 1]], sem).wait()

        pltpu.make_async_copy(
            ones_v, deg_sh.at[dstv.at[WPS - 1]], sem).wait()

    plsc.subcore_barrier()
    pltpu.sync_copy(deg_sh.at[pl.ds(row0, ROWS_PER_SUB)],
                    deg_out.at[c, pl.ds(row0, ROWS_PER_SUB)])


def _make_sc_deg():
    return pl.kernel(
        _sc_deg_body,
        out_type=jax.ShapeDtypeStruct((NC, NPAD, D), jnp.float32),
        mesh=_sc_mesh(),
        scratch_types=[
            pltpu.VMEM_SHARED((NPAD, D), jnp.float32),
            pltpu.VMEM((WPS, WIN), jnp.int32),
            pltpu.VMEM((WIN, D), jnp.float32),
            pltpu.SemaphoreType.DMA,
        ],
    )


def _dense_block(x, p0, p1, rdeg, ws, bs, wn, bn):
    agg = (p0 + p1) * rdeg
    h = (jnp.dot(x, ws, preferred_element_type=jnp.float32) + bs
         + jnp.dot(agg, wn, preferred_element_type=jnp.float32) + bn)
    return jnp.maximum(h, 0.0)


def _tc1_body(x_ref, p_ref, dp_ref, ws_ref, bs_ref, wn_ref, bn_ref,
              h_ref, rdeg_ref):
    d = dp_ref[0, :, 0:1] + dp_ref[1, :, 0:1]
    rdeg = 1.0 / jnp.maximum(d, 1.0)
    rdeg_ref[...] = rdeg
    h_ref[...] = _dense_block(x_ref[...], p_ref[0], p_ref[1], rdeg,
                              ws_ref[...], bs_ref[...], wn_ref[...], bn_ref[...])


def _tc2_body(x_ref, p_ref, rdeg_ref, ws_ref, bs_ref, wn_ref, bn_ref, h_ref):
    h_ref[...] = _dense_block(x_ref[...], p_ref[0], p_ref[1], rdeg_ref[...],
                              ws_ref[...], bs_ref[...], wn_ref[...], bn_ref[...])


def _tc3_body(x_ref, p_ref, rdeg_ref, bv_ref, ws_ref, bs_ref, wn_ref, bn_ref,
              out_ref, acc_ref, cnt_ref):
    i = pl.program_id(0)

    @pl.when(i == 0)
    def _():
        acc_ref[...] = jnp.zeros((B_GRAPHS, D), jnp.float32)
        cnt_ref[...] = jnp.zeros((B_GRAPHS, D), jnp.float32)

    h = _dense_block(x_ref[...], p_ref[0], p_ref[1], rdeg_ref[...],
                     ws_ref[...], bs_ref[...], wn_ref[...], bn_ref[...])
    bv = bv_ref[0]
    onehot = (bv == lax.broadcasted_iota(jnp.int32, (B_GRAPHS, R), 0)
              ).astype(jnp.float32)
    acc_ref[...] += jnp.dot(onehot, h, preferred_element_type=jnp.float32)
    cnt_ref[...] += jnp.broadcast_to(
        jnp.sum(onehot, axis=1, keepdims=True), (B_GRAPHS, D))

    @pl.when(i == pl.num_programs(0) - 1)
    def _():
        out_ref[...] = acc_ref[...] / jnp.maximum(cnt_ref[...], 1.0)


def _full(shape):
    return pl.BlockSpec(shape, lambda i: tuple(0 for _ in shape))


_rows = pl.BlockSpec((R, D), lambda i: (i, 0))
_p_spec = pl.BlockSpec((NC, R, D), lambda i: (0, i, 0))
_rdeg_spec = pl.BlockSpec((R, 1), lambda i: (i, 0))
_w_specs = [_full((D, D)), _full((1, D)), _full((D, D)), _full((1, D))]


def _tc1(x, aggp, degp, ws, bs, wn, bn):
    return pl.pallas_call(
        _tc1_body,
        grid=(GRID,),
        in_specs=[_rows, _p_spec, _p_spec, *_w_specs],
        out_specs=[_rows, _rdeg_spec],
        out_shape=[jax.ShapeDtypeStruct((N_NODES, D), jnp.float32),
                   jax.ShapeDtypeStruct((N_NODES, 1), jnp.float32)],
    )(x, aggp, degp, ws, bs, wn, bn)


def _tc2(x, aggp, rdeg, ws, bs, wn, bn):
    return pl.pallas_call(
        _tc2_body,
        grid=(GRID,),
        in_specs=[_rows, _p_spec, _rdeg_spec, *_w_specs],
        out_specs=_rows,
        out_shape=jax.ShapeDtypeStruct((N_NODES, D), jnp.float32),
    )(x, aggp, rdeg, ws, bs, wn, bn)


def _tc3(x, aggp, rdeg, bv3, ws, bs, wn, bn):
    return pl.pallas_call(
        _tc3_body,
        grid=(GRID,),
        in_specs=[_rows, _p_spec, _rdeg_spec,
                  pl.BlockSpec((1, 1, R), lambda i: (i, 0, 0)), *_w_specs],
        out_specs=_full((B_GRAPHS, D)),
        out_shape=jax.ShapeDtypeStruct((B_GRAPHS, D), jnp.float32),
        scratch_shapes=[pltpu.VMEM((B_GRAPHS, D), jnp.float32),
                        pltpu.VMEM((B_GRAPHS, D), jnp.float32)],
    )(x, aggp, rdeg, bv3, ws, bs, wn, bn)


def kernel(x, edge_index, batch_vec,
           W_self0, b_self0, W_neigh0, b_neigh0,
           W_self1, b_self1, W_neigh1, b_neigh1,
           W_self2, b_self2, W_neigh2, b_neigh2):
    pad = E_PAD - N_EDGES
    pad_idx = jnp.arange(pad, dtype=jnp.int32)
    src = jnp.concatenate([edge_index[0], pad_idx % N_NODES])
    dst = jnp.concatenate([edge_index[1], N_NODES + pad_idx % (NPAD - N_NODES)])
    src2 = src.reshape(E_PAD // WIN, WIN)
    dst2 = dst.reshape(E_PAD // WIN, WIN)
    z2d = jnp.zeros((NPAD, D), jnp.float32)
    ones2d = jnp.ones((WIN, D), jnp.float32)
    bv3 = batch_vec.reshape(GRID, 1, R)

    sc_agg = _make_sc_agg()
    sc_deg = _make_sc_deg()

    degp = sc_deg(dst2, z2d, ones2d)
    aggp = sc_agg(x, src2, dst2, z2d)
    h1, rdeg = _tc1(x, aggp, degp,
                    W_self0, b_self0.reshape(1, D), W_neigh0, b_neigh0.reshape(1, D))
    aggp = sc_agg(h1, src2, dst2, z2d)
    h2 = _tc2(h1, aggp, rdeg,
              W_self1, b_self1.reshape(1, D), W_neigh1, b_neigh1.reshape(1, D))
    aggp = sc_agg(h2, src2, dst2, z2d)
    out = _tc3(h2, aggp, rdeg, bv3,
               W_self2, b_self2.reshape(1, D), W_neigh2, b_neigh2.reshape(1, D))
    return out


# speedup vs baseline: 1.2720x; 1.2720x over previous
import functools

import jax
import jax.numpy as jnp
from jax import lax
from jax.experimental import pallas as pl
from jax.experimental.pallas import tpu as pltpu
from jax.experimental.pallas import tpu_sc as plsc

N_NODES = 10000
N_EDGES = 320000
D = 128
B_GRAPHS = 64

NC, NS = 2, 16
NW = NC * NS
WIN = 128
NWIN = 80
NSTAGE = 2
WPS = NWIN // NSTAGE
EDGES_PER_TILE = WIN * NWIN
E_PAD = EDGES_PER_TILE * NW
NPAD = 10112
ROWS_PER_SUB = NPAD // NS
R = 1000
GRID = N_NODES // R


def _sc_mesh():
    return plsc.VectorSubcoreMesh(
        core_axis_name="c", subcore_axis_name="s", num_cores=NC, num_subcores=NS
    )


def _sc_agg_body(x_hbm, src_hbm, dst_hbm, z2d_hbm, agg_out,
                 agg_sh, srcv, dstv, buf_a, buf_b,
                 sem_a, sem_b, sem_sa, sem_sb):
    c = lax.axis_index("c")
    s = lax.axis_index("s")
    w = s * NC + c
    row0 = s * ROWS_PER_SUB

    pltpu.sync_copy(z2d_hbm.at[pl.ds(row0, ROWS_PER_SUB)],
                    agg_sh.at[pl.ds(row0, ROWS_PER_SUB)])
    plsc.subcore_barrier()

    for t in range(NSTAGE):
        wrow = w * NWIN + t * WPS
        pltpu.sync_copy(src_hbm.at[pl.ds(wrow, WPS)], srcv)
        pltpu.sync_copy(dst_hbm.at[pl.ds(wrow, WPS)], dstv)

        pltpu.async_copy(x_hbm.at[srcv.at[0]], buf_a, sem_a)
        pltpu.async_copy(x_hbm.at[srcv.at[1]], buf_b, sem_b)

        @pl.loop(0, WPS, step=2)
        def _(j):
            pltpu.make_async_copy(x_hbm.at[srcv.at[j]], buf_a, sem_a).wait()
            pltpu.sync_copy(buf_a, agg_sh.at[dstv.at[j]], add=True)

            @pl.when(j + 2 < WPS)
            def _():
                pltpu.async_copy(x_hbm.at[srcv.at[j + 2]], buf_a, sem_a)

            pltpu.make_async_copy(x_hbm.at[srcv.at[j + 1]], buf_b, sem_b).wait()
            pltpu.sync_copy(buf_b, agg_sh.at[dstv.at[j + 1]], add=True)

            @pl.when(j + 3 < WPS)
            def _():
                pltpu.async_copy(x_hbm.at[srcv.at[j + 3]], buf_b, sem_b)

    plsc.subcore_barrier()

    pltpu.sync_copy(agg_sh.at[pl.ds(row0, ROWS_PER_SUB)],
                    agg_out.at[c, pl.ds(row0, ROWS_PER_SUB)])


def _make_sc_agg():
    return pl.kernel(
        _sc_agg_body,
        out_type=jax.ShapeDtypeStruct((NC, NPAD, D), jnp.float32),
        mesh=_sc_mesh(),
        scratch_types=[
            pltpu.VMEM_SHARED((NPAD, D), jnp.float32),
            pltpu.VMEM((WPS, WIN), jnp.int32),
            pltpu.VMEM((WPS, WIN), jnp.int32),
            pltpu.VMEM((WIN, D), jnp.float32),
            pltpu.VMEM((WIN, D), jnp.float32),
            pltpu.SemaphoreType.DMA,
            pltpu.SemaphoreType.DMA,
            pltpu.SemaphoreType.DMA,
            pltpu.SemaphoreType.DMA,
        ],
    )


def _sc_deg_body(dst_hbm, z2d_hbm, ones_hbm, deg_out,
                 deg_sh, dstv, ones_v, sem):
    c = lax.axis_index("c")
    s = lax.axis_index("s")
    w = s * NC + c
    row0 = s * ROWS_PER_SUB

    pltpu.sync_copy(z2d_hbm.at[pl.ds(row0, ROWS_PER_SUB)],
                    deg_sh.at[pl.ds(row0, ROWS_PER_SUB)])
    pltpu.sync_copy(ones_hbm, ones_v)
    plsc.subcore_barrier()

    for t in range(NSTAGE):
        wrow = w * NWIN + t * WPS
        pltpu.sync_copy(dst_hbm.at[pl.ds(wrow, WPS)], dstv)

        pltpu.async_copy(ones_v, deg_sh.at[dstv.at[0]], sem, add=True)

        @pl.loop(1, WPS)
        def _(j):
            pltpu.async_copy(ones_v, deg_sh.at[dstv.at[j]], sem, add=True)
            pltpu.make_async_copy(ones_v, deg_sh.at[dstv.at[j - 1]], sem).wait()

        pltpu.make_async_copy(
            ones_v, deg_sh.at[dstv.at[WPS - 1]], sem).wait()

    plsc.subcore_barrier()
    pltpu.sync_copy(deg_sh.at[pl.ds(row0, ROWS_PER_SUB)],
                    deg_out.at[c, pl.ds(row0, ROWS_PER_SUB)])


def _make_sc_deg():
    return pl.kernel(
        _sc_deg_body,
        out_type=jax.ShapeDtypeStruct((NC, NPAD, D), jnp.float32),
        mesh=_sc_mesh(),
        scratch_types=[
            pltpu.VMEM_SHARED((NPAD, D), jnp.float32),
            pltpu.VMEM((WPS, WIN), jnp.int32),
            pltpu.VMEM((WIN, D), jnp.float32),
            pltpu.SemaphoreType.DMA,
        ],
    )


def _dense_block(x, p0, p1, rdeg, ws, bs, wn, bn):
    agg = (p0 + p1) * rdeg
    h = (jnp.dot(x, ws, preferred_element_type=jnp.float32) + bs
         + jnp.dot(agg, wn, preferred_element_type=jnp.float32) + bn)
    return jnp.maximum(h, 0.0)


def _tc1_body(x_ref, p_ref, dp_ref, ws_ref, bs_ref, wn_ref, bn_ref,
              h_ref, rdeg_ref):
    d = dp_ref[0, :, 0:1] + dp_ref[1, :, 0:1]
    rdeg = 1.0 / jnp.maximum(d, 1.0)
    rdeg_ref[...] = rdeg
    h_ref[...] = _dense_block(x_ref[...], p_ref[0], p_ref[1], rdeg,
                              ws_ref[...], bs_ref[...], wn_ref[...], bn_ref[...])


def _tc2_body(x_ref, p_ref, rdeg_ref, ws_ref, bs_ref, wn_ref, bn_ref, h_ref):
    h_ref[...] = _dense_block(x_ref[...], p_ref[0], p_ref[1], rdeg_ref[...],
                              ws_ref[...], bs_ref[...], wn_ref[...], bn_ref[...])


def _tc3_body(x_ref, p_ref, rdeg_ref, bv_ref, ws_ref, bs_ref, wn_ref, bn_ref,
              out_ref, acc_ref, cnt_ref):
    i = pl.program_id(0)

    @pl.when(i == 0)
    def _():
        acc_ref[...] = jnp.zeros((B_GRAPHS, D), jnp.float32)
        cnt_ref[...] = jnp.zeros((B_GRAPHS, D), jnp.float32)

    h = _dense_block(x_ref[...], p_ref[0], p_ref[1], rdeg_ref[...],
                     ws_ref[...], bs_ref[...], wn_ref[...], bn_ref[...])
    bv = bv_ref[0]
    onehot = (bv == lax.broadcasted_iota(jnp.int32, (B_GRAPHS, R), 0)
              ).astype(jnp.float32)
    acc_ref[...] += jnp.dot(onehot, h, preferred_element_type=jnp.float32)
    cnt_ref[...] += jnp.broadcast_to(
        jnp.sum(onehot, axis=1, keepdims=True), (B_GRAPHS, D))

    @pl.when(i == pl.num_programs(0) - 1)
    def _():
        out_ref[...] = acc_ref[...] / jnp.maximum(cnt_ref[...], 1.0)


def _full(shape):
    return pl.BlockSpec(shape, lambda i: tuple(0 for _ in shape))


_rows = pl.BlockSpec((R, D), lambda i: (i, 0))
_p_spec = pl.BlockSpec((NC, R, D), lambda i: (0, i, 0))
_rdeg_spec = pl.BlockSpec((R, 1), lambda i: (i, 0))
_w_specs = [_full((D, D)), _full((1, D)), _full((D, D)), _full((1, D))]


def _tc1(x, aggp, degp, ws, bs, wn, bn):
    return pl.pallas_call(
        _tc1_body,
        grid=(GRID,),
        in_specs=[_rows, _p_spec, _p_spec, *_w_specs],
        out_specs=[_rows, _rdeg_spec],
        out_shape=[jax.ShapeDtypeStruct((N_NODES, D), jnp.float32),
                   jax.ShapeDtypeStruct((N_NODES, 1), jnp.float32)],
    )(x, aggp, degp, ws, bs, wn, bn)


def _tc2(x, aggp, rdeg, ws, bs, wn, bn):
    return pl.pallas_call(
        _tc2_body,
        grid=(GRID,),
        in_specs=[_rows, _p_spec, _rdeg_spec, *_w_specs],
        out_specs=_rows,
        out_shape=jax.ShapeDtypeStruct((N_NODES, D), jnp.float32),
    )(x, aggp, rdeg, ws, bs, wn, bn)


def _tc3(x, aggp, rdeg, bv3, ws, bs, wn, bn):
    return pl.pallas_call(
        _tc3_body,
        grid=(GRID,),
        in_specs=[_rows, _p_spec, _rdeg_spec,
                  pl.BlockSpec((1, 1, R), lambda i: (i, 0, 0)), *_w_specs],
        out_specs=_full((B_GRAPHS, D)),
        out_shape=jax.ShapeDtypeStruct((B_GRAPHS, D), jnp.float32),
        scratch_shapes=[pltpu.VMEM((B_GRAPHS, D), jnp.float32),
                        pltpu.VMEM((B_GRAPHS, D), jnp.float32)],
    )(x, aggp, rdeg, bv3, ws, bs, wn, bn)


def kernel(x, edge_index, batch_vec,
           W_self0, b_self0, W_neigh0, b_neigh0,
           W_self1, b_self1, W_neigh1, b_neigh1,
           W_self2, b_self2, W_neigh2, b_neigh2):
    pad = E_PAD - N_EDGES
    pad_idx = jnp.arange(pad, dtype=jnp.int32)
    src = jnp.concatenate([edge_index[0], pad_idx % N_NODES])
    dst = jnp.concatenate([edge_index[1], N_NODES + pad_idx % (NPAD - N_NODES)])
    src2 = src.reshape(E_PAD // WIN, WIN)
    dst2 = dst.reshape(E_PAD // WIN, WIN)
    z2d = jnp.zeros((NPAD, D), jnp.float32)
    ones2d = jnp.ones((WIN, D), jnp.float32)
    bv3 = batch_vec.reshape(GRID, 1, R)

    sc_agg = _make_sc_agg()
    sc_deg = _make_sc_deg()

    degp = sc_deg(dst2, z2d, ones2d)
    aggp = sc_agg(x, src2, dst2, z2d)
    h1, rdeg = _tc1(x, aggp, degp,
                    W_self0, b_self0.reshape(1, D), W_neigh0, b_neigh0.reshape(1, D))
    aggp = sc_agg(h1, src2, dst2, z2d)
    h2 = _tc2(h1, aggp, rdeg,
              W_self1, b_self1.reshape(1, D), W_neigh1, b_neigh1.reshape(1, D))
    aggp = sc_agg(h2, src2, dst2, z2d)
    out = _tc3(h2, aggp, rdeg, bv3,
               W_self2, b_self2.reshape(1, D), W_neigh2, b_neigh2.reshape(1, D))
    return out


# speedup vs baseline: 1.3924x; 1.0947x over previous
import dataclasses
import functools

import jax
import jax.numpy as jnp
from jax import lax
from jax.experimental import pallas as pl
from jax.experimental.pallas import tpu as pltpu
from jax.experimental.pallas import tpu_sc as plsc

N_NODES = 10000
N_EDGES = 320000
D = 128
B_GRAPHS = 64

NC, NS = 2, 16
NW = NC * NS
WIN = 128
NWIN = 80
NSTAGE = 5
WPS = NWIN // NSTAGE
EDGES_PER_TILE = WIN * NWIN
E_PAD = EDGES_PER_TILE * NW
NPAD = 10112
ROWS_PER_SUB = NPAD // NS
R = 1000
GRID = N_NODES // R


def _sc_mesh():
    return plsc.VectorSubcoreMesh(
        core_axis_name="c", subcore_axis_name="s", num_cores=NC, num_subcores=NS
    )


def _sc_agg_body(with_deg, *refs):
    if with_deg:
        (x_hbm, src_hbm, dst_hbm, z2d_hbm, z1d_hbm, agg_out, deg_out,
         agg_sh, srcv, dstv, buf_a, buf_b, deg_v,
         sem_a, sem_b) = refs
    else:
        (x_hbm, src_hbm, dst_hbm, z2d_hbm, agg_out,
         agg_sh, srcv, dstv, buf_a, buf_b, sem_a, sem_b) = refs

    c = lax.axis_index("c")
    s = lax.axis_index("s")
    w = s * NC + c
    row0 = s * ROWS_PER_SUB

    pltpu.sync_copy(z2d_hbm.at[pl.ds(row0, ROWS_PER_SUB)],
                    agg_sh.at[pl.ds(row0, ROWS_PER_SUB)])
    if with_deg:
        pltpu.sync_copy(z1d_hbm, deg_v)
    plsc.subcore_barrier()

    def deg_window(j):
        for k in range(WIN // 16):
            d = dstv[j, pl.ds(k * 16, 16)]
            cnt, m = plsc.scan_count(d)
            plsc.addupdate_scatter(deg_v, [d], cnt.astype(jnp.float32),
                                   mask=m)

    for t in range(NSTAGE):
        wrow = w * NWIN + t * WPS
        pltpu.sync_copy(src_hbm.at[pl.ds(wrow, WPS)], srcv)
        pltpu.sync_copy(dst_hbm.at[pl.ds(wrow, WPS)], dstv)

        pltpu.async_copy(x_hbm.at[srcv.at[0]], buf_a, sem_a)
        pltpu.async_copy(x_hbm.at[srcv.at[1]], buf_b, sem_b)

        @pl.loop(0, WPS, step=2)
        def _(j):
            if with_deg:
                deg_window(j)
            pltpu.make_async_copy(x_hbm.at[srcv.at[j]], buf_a, sem_a).wait()
            pltpu.sync_copy(buf_a, agg_sh.at[dstv.at[j]], add=True)

            @pl.when(j + 2 < WPS)
            def _():
                pltpu.async_copy(x_hbm.at[srcv.at[j + 2]], buf_a, sem_a)

            if with_deg:
                deg_window(j + 1)
            pltpu.make_async_copy(x_hbm.at[srcv.at[j + 1]], buf_b, sem_b).wait()
            pltpu.sync_copy(buf_b, agg_sh.at[dstv.at[j + 1]], add=True)

            @pl.when(j + 3 < WPS)
            def _():
                pltpu.async_copy(x_hbm.at[srcv.at[j + 3]], buf_b, sem_b)

    plsc.subcore_barrier()

    pltpu.sync_copy(agg_sh.at[pl.ds(row0, ROWS_PER_SUB)],
                    agg_out.at[c, pl.ds(row0, ROWS_PER_SUB)])
    if with_deg:
        pltpu.sync_copy(deg_v, deg_out.at[pl.ds(w * NPAD, NPAD)])


def _make_sc_agg(with_deg):
    out_type = [jax.ShapeDtypeStruct((NC, NPAD, D), jnp.float32)]
    scratch = [
        pltpu.VMEM_SHARED((NPAD, D), jnp.float32),
        pltpu.VMEM((WPS, WIN), jnp.int32),
        pltpu.VMEM((WPS, WIN), jnp.int32),
        pltpu.VMEM((WIN, D), jnp.float32),
        pltpu.VMEM((WIN, D), jnp.float32),
    ]
    if with_deg:
        out_type.append(jax.ShapeDtypeStruct((NW * NPAD,), jnp.float32))
        scratch.append(pltpu.VMEM((NPAD,), jnp.float32))
    scratch += [pltpu.SemaphoreType.DMA, pltpu.SemaphoreType.DMA]
    cp = pltpu.CompilerParams()
    if with_deg and "needs_layout_passes" in pltpu.CompilerParams.__dataclass_fields__:
        cp = dataclasses.replace(cp, needs_layout_passes=False)
    return pl.kernel(
        functools.partial(_sc_agg_body, with_deg),
        out_type=tuple(out_type) if with_deg else out_type[0],
        mesh=_sc_mesh(),
        scratch_types=scratch,
        compiler_params=cp,
    )


def _dense_block(x, p0, p1, rdeg, ws, bs, wn, bn):
    agg = (p0 + p1) * rdeg
    h = (jnp.dot(x, ws, preferred_element_type=jnp.float32) + bs
         + jnp.dot(agg, wn, preferred_element_type=jnp.float32) + bn)
    return jnp.maximum(h, 0.0)


def _tc1_body(x_ref, p_ref, dp_ref, ws_ref, bs_ref, wn_ref, bn_ref,
              h_ref, rdeg_ref):
    d = lax.dot_general(dp_ref[0], jnp.ones((NW, 1), jnp.float32),
                        (((0,), (0,)), ((), ())),
                        preferred_element_type=jnp.float32)
    rdeg = 1.0 / jnp.maximum(d, 1.0)
    rdeg_ref[...] = rdeg
    h_ref[...] = _dense_block(x_ref[...], p_ref[0], p_ref[1], rdeg,
                              ws_ref[...], bs_ref[...], wn_ref[...], bn_ref[...])


def _tc2_body(x_ref, p_ref, rdeg_ref, ws_ref, bs_ref, wn_ref, bn_ref, h_ref):
    h_ref[...] = _dense_block(x_ref[...], p_ref[0], p_ref[1], rdeg_ref[...],
                              ws_ref[...], bs_ref[...], wn_ref[...], bn_ref[...])


def _tc3_body(x_ref, p_ref, rdeg_ref, bv_ref, ws_ref, bs_ref, wn_ref, bn_ref,
              out_ref, acc_ref, cnt_ref):
    i = pl.program_id(0)

    @pl.when(i == 0)
    def _():
        acc_ref[...] = jnp.zeros((B_GRAPHS, D), jnp.float32)
        cnt_ref[...] = jnp.zeros((B_GRAPHS, D), jnp.float32)

    h = _dense_block(x_ref[...], p_ref[0], p_ref[1], rdeg_ref[...],
                     ws_ref[...], bs_ref[...], wn_ref[...], bn_ref[...])
    bv = bv_ref[0]
    onehot = (bv == lax.broadcasted_iota(jnp.int32, (B_GRAPHS, R), 0)
              ).astype(jnp.float32)
    acc_ref[...] += jnp.dot(onehot, h, preferred_element_type=jnp.float32)
    cnt_ref[...] += jnp.broadcast_to(
        jnp.sum(onehot, axis=1, keepdims=True), (B_GRAPHS, D))

    @pl.when(i == pl.num_programs(0) - 1)
    def _():
        out_ref[...] = acc_ref[...] / jnp.maximum(cnt_ref[...], 1.0)


def _full(shape):
    return pl.BlockSpec(shape, lambda i: tuple(0 for _ in shape))


_rows = pl.BlockSpec((R, D), lambda i: (i, 0))
_p_spec = pl.BlockSpec((NC, R, D), lambda i: (0, i, 0))
_rdeg_spec = pl.BlockSpec((R, 1), lambda i: (i, 0))
_w_specs = [_full((D, D)), _full((1, D)), _full((D, D)), _full((1, D))]


def _tc1(x, aggp, degp, ws, bs, wn, bn):
    return pl.pallas_call(
        _tc1_body,
        grid=(GRID,),
        in_specs=[_rows, _p_spec, pl.BlockSpec((1, NW, R), lambda i: (i, 0, 0)),
                  *_w_specs],
        out_specs=[_rows, _rdeg_spec],
        out_shape=[jax.ShapeDtypeStruct((N_NODES, D), jnp.float32),
                   jax.ShapeDtypeStruct((N_NODES, 1), jnp.float32)],
    )(x, aggp, degp, ws, bs, wn, bn)


def _tc2(x, aggp, rdeg, ws, bs, wn, bn):
    return pl.pallas_call(
        _tc2_body,
        grid=(GRID,),
        in_specs=[_rows, _p_spec, _rdeg_spec, *_w_specs],
        out_specs=_rows,
        out_shape=jax.ShapeDtypeStruct((N_NODES, D), jnp.float32),
    )(x, aggp, rdeg, ws, bs, wn, bn)


def _tc3(x, aggp, rdeg, bv3, ws, bs, wn, bn):
    return pl.pallas_call(
        _tc3_body,
        grid=(GRID,),
        in_specs=[_rows, _p_spec, _rdeg_spec,
                  pl.BlockSpec((1, 1, R), lambda i: (i, 0, 0)), *_w_specs],
        out_specs=_full((B_GRAPHS, D)),
        out_shape=jax.ShapeDtypeStruct((B_GRAPHS, D), jnp.float32),
        scratch_shapes=[pltpu.VMEM((B_GRAPHS, D), jnp.float32),
                        pltpu.VMEM((B_GRAPHS, D), jnp.float32)],
    )(x, aggp, rdeg, bv3, ws, bs, wn, bn)


def kernel(x, edge_index, batch_vec,
           W_self0, b_self0, W_neigh0, b_neigh0,
           W_self1, b_self1, W_neigh1, b_neigh1,
           W_self2, b_self2, W_neigh2, b_neigh2):
    pad = E_PAD - N_EDGES
    pad_idx = jnp.arange(pad, dtype=jnp.int32)
    src = jnp.concatenate([edge_index[0], pad_idx % N_NODES])
    dst = jnp.concatenate([edge_index[1], N_NODES + pad_idx % (NPAD - N_NODES)])
    src2 = src.reshape(E_PAD // WIN, WIN)
    dst2 = dst.reshape(E_PAD // WIN, WIN)
    z2d = jnp.zeros((NPAD, D), jnp.float32)
    z1d = jnp.zeros((NPAD,), jnp.float32)
    bv3 = batch_vec.reshape(GRID, 1, R)

    sc_agg_deg = _make_sc_agg(True)
    sc_agg = _make_sc_agg(False)

    aggp, degp = sc_agg_deg(x, src2, dst2, z2d, z1d)
    dp3 = (degp.reshape(NW, NPAD)[:, :N_NODES]
           .reshape(NW, GRID, R).transpose(1, 0, 2))
    h1, rdeg = _tc1(x, aggp, dp3,
                    W_self0, b_self0.reshape(1, D), W_neigh0, b_neigh0.reshape(1, D))
    aggp = sc_agg(h1, src2, dst2, z2d)
    h2 = _tc2(h1, aggp, rdeg,
              W_self1, b_self1.reshape(1, D), W_neigh1, b_neigh1.reshape(1, D))
    aggp = sc_agg(h2, src2, dst2, z2d)
    out = _tc3(h2, aggp, rdeg, bv3,
               W_self2, b_self2.reshape(1, D), W_neigh2, b_neigh2.reshape(1, D))
    return out


# speedup vs baseline: 1.4857x; 1.0669x over previous
import dataclasses
import functools

import jax
import jax.numpy as jnp
from jax import lax
from jax.experimental import pallas as pl
from jax.experimental.pallas import tpu as pltpu
from jax.experimental.pallas import tpu_sc as plsc

N_NODES = 10000
N_EDGES = 320000
D = 128
B_GRAPHS = 64

NC, NS = 2, 16
NW = NC * NS
WIN = 128
NWIN = 80
NSTAGE_DEG, NSTAGE_PLAIN = 5, 2
EDGES_PER_TILE = WIN * NWIN
E_PAD = EDGES_PER_TILE * NW
NPAD = 10112
ROWS_PER_SUB = NPAD // NS
R = 2000
GRID = N_NODES // R


def _sc_mesh():
    return plsc.VectorSubcoreMesh(
        core_axis_name="c", subcore_axis_name="s", num_cores=NC, num_subcores=NS
    )


def _sc_agg_body(with_deg, *refs):
    if with_deg:
        (x_hbm, src_hbm, dst_hbm, z2d_hbm, z1d_hbm, agg_out, deg_out,
         agg_sh, srcv, dstv, buf_a, buf_b, deg_v,
         sem_a, sem_b) = refs
    else:
        (x_hbm, src_hbm, dst_hbm, z2d_hbm, agg_out,
         agg_sh, srcv, dstv, buf_a, buf_b, sem_a, sem_b) = refs

    nstage = NSTAGE_DEG if with_deg else NSTAGE_PLAIN
    wps = NWIN // nstage
    c = lax.axis_index("c")
    s = lax.axis_index("s")
    w = s * NC + c
    row0 = s * ROWS_PER_SUB

    pltpu.sync_copy(z2d_hbm.at[pl.ds(row0, ROWS_PER_SUB)],
                    agg_sh.at[pl.ds(row0, ROWS_PER_SUB)])
    if with_deg:
        pltpu.sync_copy(z1d_hbm, deg_v)
    plsc.subcore_barrier()

    def deg_window(j):
        for k in range(WIN // 16):
            d = dstv[j, pl.ds(k * 16, 16)]
            cnt, m = plsc.scan_count(d)
            plsc.addupdate_scatter(deg_v, [d], cnt.astype(jnp.float32),
                                   mask=m)

    for t in range(nstage):
        wrow = w * NWIN + t * wps
        pltpu.sync_copy(src_hbm.at[pl.ds(wrow, wps)], srcv)
        pltpu.sync_copy(dst_hbm.at[pl.ds(wrow, wps)], dstv)

        pltpu.async_copy(x_hbm.at[srcv.at[0]], buf_a, sem_a)
        pltpu.async_copy(x_hbm.at[srcv.at[1]], buf_b, sem_b)

        @pl.loop(0, wps, step=2)
        def _(j):
            if with_deg:
                deg_window(j)
            pltpu.make_async_copy(x_hbm.at[srcv.at[j]], buf_a, sem_a).wait()
            pltpu.sync_copy(buf_a, agg_sh.at[dstv.at[j]], add=True)

            @pl.when(j + 2 < wps)
            def _():
                pltpu.async_copy(x_hbm.at[srcv.at[j + 2]], buf_a, sem_a)

            if with_deg:
                deg_window(j + 1)
            pltpu.make_async_copy(x_hbm.at[srcv.at[j + 1]], buf_b, sem_b).wait()
            pltpu.sync_copy(buf_b, agg_sh.at[dstv.at[j + 1]], add=True)

            @pl.when(j + 3 < wps)
            def _():
                pltpu.async_copy(x_hbm.at[srcv.at[j + 3]], buf_b, sem_b)

    plsc.subcore_barrier()

    pltpu.sync_copy(agg_sh.at[pl.ds(row0, ROWS_PER_SUB)],
                    agg_out.at[c, pl.ds(row0, ROWS_PER_SUB)])
    if with_deg:
        pltpu.sync_copy(deg_v, deg_out.at[pl.ds(w * NPAD, NPAD)])


def _make_sc_agg(with_deg):
    wps = NWIN // (NSTAGE_DEG if with_deg else NSTAGE_PLAIN)
    out_type = [jax.ShapeDtypeStruct((NC, NPAD, D), jnp.float32)]
    scratch = [
        pltpu.VMEM_SHARED((NPAD, D), jnp.float32),
        pltpu.VMEM((wps, WIN), jnp.int32),
        pltpu.VMEM((wps, WIN), jnp.int32),
        pltpu.VMEM((WIN, D), jnp.float32),
        pltpu.VMEM((WIN, D), jnp.float32),
    ]
    if with_deg:
        out_type.append(jax.ShapeDtypeStruct((NW * NPAD,), jnp.float32))
        scratch.append(pltpu.VMEM((NPAD,), jnp.float32))
    scratch += [pltpu.SemaphoreType.DMA, pltpu.SemaphoreType.DMA]
    cp = pltpu.CompilerParams()
    if with_deg and "needs_layout_passes" in pltpu.CompilerParams.__dataclass_fields__:
        cp = dataclasses.replace(cp, needs_layout_passes=False)
    return pl.kernel(
        functools.partial(_sc_agg_body, with_deg),
        out_type=tuple(out_type) if with_deg else out_type[0],
        mesh=_sc_mesh(),
        scratch_types=scratch,
        compiler_params=cp,
    )


def _dense_block(x, p0, p1, rdeg, ws, bs, wn, bn):
    agg = (p0 + p1) * rdeg
    h = (jnp.dot(x, ws, preferred_element_type=jnp.float32) + bs
         + jnp.dot(agg, wn, preferred_element_type=jnp.float32) + bn)
    return jnp.maximum(h, 0.0)


def _tc1_body(x_ref, p_ref, dp_ref, ws_ref, bs_ref, wn_ref, bn_ref,
              h_ref, rdeg_ref):
    d = lax.dot_general(dp_ref[0], jnp.ones((NW, 1), jnp.float32),
                        (((0,), (0,)), ((), ())),
                        preferred_element_type=jnp.float32)
    rdeg = 1.0 / jnp.maximum(d, 1.0)
    rdeg_ref[...] = rdeg
    h_ref[...] = _dense_block(x_ref[...], p_ref[0], p_ref[1], rdeg,
                              ws_ref[...], bs_ref[...], wn_ref[...], bn_ref[...])


def _tc2_body(x_ref, p_ref, rdeg_ref, ws_ref, bs_ref, wn_ref, bn_ref, h_ref):
    h_ref[...] = _dense_block(x_ref[...], p_ref[0], p_ref[1], rdeg_ref[...],
                              ws_ref[...], bs_ref[...], wn_ref[...], bn_ref[...])


def _tc3_body(x_ref, p_ref, rdeg_ref, bv_ref, ws_ref, bs_ref, wn_ref, bn_ref,
              out_ref, acc_ref, cnt_ref):
    i = pl.program_id(0)

    @pl.when(i == 0)
    def _():
        acc_ref[...] = jnp.zeros((B_GRAPHS, D), jnp.float32)
        cnt_ref[...] = jnp.zeros((B_GRAPHS, D), jnp.float32)

    h = _dense_block(x_ref[...], p_ref[0], p_ref[1], rdeg_ref[...],
                     ws_ref[...], bs_ref[...], wn_ref[...], bn_ref[...])
    bv = bv_ref[0]
    onehot = (bv == lax.broadcasted_iota(jnp.int32, (B_GRAPHS, R), 0)
              ).astype(jnp.float32)
    acc_ref[...] += jnp.dot(onehot, h, preferred_element_type=jnp.float32)
    cnt_ref[...] += jnp.broadcast_to(
        jnp.sum(onehot, axis=1, keepdims=True), (B_GRAPHS, D))

    @pl.when(i == pl.num_programs(0) - 1)
    def _():
        out_ref[...] = acc_ref[...] / jnp.maximum(cnt_ref[...], 1.0)


def _full(shape):
    return pl.BlockSpec(shape, lambda i: tuple(0 for _ in shape))


_rows = pl.BlockSpec((R, D), lambda i: (i, 0))
_p_spec = pl.BlockSpec((NC, R, D), lambda i: (0, i, 0))
_rdeg_spec = pl.BlockSpec((R, 1), lambda i: (i, 0))
_w_specs = [_full((D, D)), _full((1, D)), _full((D, D)), _full((1, D))]


def _tc1(x, aggp, degp, ws, bs, wn, bn):
    return pl.pallas_call(
        _tc1_body,
        grid=(GRID,),
        in_specs=[_rows, _p_spec, pl.BlockSpec((1, NW, R), lambda i: (i, 0, 0)),
                  *_w_specs],
        out_specs=[_rows, _rdeg_spec],
        out_shape=[jax.ShapeDtypeStruct((N_NODES, D), jnp.float32),
                   jax.ShapeDtypeStruct((N_NODES, 1), jnp.float32)],
    )(x, aggp, degp, ws, bs, wn, bn)


def _tc2(x, aggp, rdeg, ws, bs, wn, bn):
    return pl.pallas_call(
        _tc2_body,
        grid=(GRID,),
        in_specs=[_rows, _p_spec, _rdeg_spec, *_w_specs],
        out_specs=_rows,
        out_shape=jax.ShapeDtypeStruct((N_NODES, D), jnp.float32),
    )(x, aggp, rdeg, ws, bs, wn, bn)


def _tc3(x, aggp, rdeg, bv3, ws, bs, wn, bn):
    return pl.pallas_call(
        _tc3_body,
        grid=(GRID,),
        in_specs=[_rows, _p_spec, _rdeg_spec,
                  pl.BlockSpec((1, 1, R), lambda i: (i, 0, 0)), *_w_specs],
        out_specs=_full((B_GRAPHS, D)),
        out_shape=jax.ShapeDtypeStruct((B_GRAPHS, D), jnp.float32),
        scratch_shapes=[pltpu.VMEM((B_GRAPHS, D), jnp.float32),
                        pltpu.VMEM((B_GRAPHS, D), jnp.float32)],
    )(x, aggp, rdeg, bv3, ws, bs, wn, bn)


def kernel(x, edge_index, batch_vec,
           W_self0, b_self0, W_neigh0, b_neigh0,
           W_self1, b_self1, W_neigh1, b_neigh1,
           W_self2, b_self2, W_neigh2, b_neigh2):
    pad = E_PAD - N_EDGES
    pad_idx = jnp.arange(pad, dtype=jnp.int32)
    src = jnp.concatenate([edge_index[0], pad_idx % N_NODES])
    dst = jnp.concatenate([edge_index[1], N_NODES + pad_idx % (NPAD - N_NODES)])
    src2 = src.reshape(E_PAD // WIN, WIN)
    dst2 = dst.reshape(E_PAD // WIN, WIN)
    z2d = jnp.zeros((NPAD, D), jnp.float32)
    z1d = jnp.zeros((NPAD,), jnp.float32)
    bv3 = batch_vec.reshape(GRID, 1, R)

    sc_agg_deg = _make_sc_agg(True)
    sc_agg = _make_sc_agg(False)

    aggp, degp = sc_agg_deg(x, src2, dst2, z2d, z1d)
    dp3 = (degp.reshape(NW, NPAD)[:, :N_NODES]
           .reshape(NW, GRID, R).transpose(1, 0, 2))
    h1, rdeg = _tc1(x, aggp, dp3,
                    W_self0, b_self0.reshape(1, D), W_neigh0, b_neigh0.reshape(1, D))
    aggp = sc_agg(h1, src2, dst2, z2d)
    h2 = _tc2(h1, aggp, rdeg,
              W_self1, b_self1.reshape(1, D), W_neigh1, b_neigh1.reshape(1, D))
    aggp = sc_agg(h2, src2, dst2, z2d)
    out = _tc3(h2, aggp, rdeg, bv3,
               W_self2, b_self2.reshape(1, D), W_neigh2, b_neigh2.reshape(1, D))
    return out
